# Initial kernel scaffold; baseline (speedup 1.0000x reference)
import jax
import jax.numpy as jnp
from jax.experimental import pallas as pl


def kernel(unknown, known, unknow_feats, known_feats):
    raise NotImplementedError("write your pallas kernel here")


# speedup vs baseline: 24.7947x; 24.7947x over previous
import functools

import jax
import jax.numpy as jnp
from jax.experimental import pallas as pl
from jax.experimental.pallas import tpu as pltpu

B, N, M, C1, C2 = 8, 4096, 1024, 64, 128
TN = 512


def _fp_kernel(unknown_ref, known_ref, unknow_feats_ref, known_feats_ref,
               out_ref):
    u = unknown_ref[0]
    k = known_ref[0]
    kf = known_feats_ref[0]
    uf = unknow_feats_ref[0]

    u2 = jnp.sum(u * u, axis=-1, keepdims=True)
    k2 = jnp.sum(k * k, axis=-1)[None, :]
    cross = jax.lax.dot_general(
        u, k, (((1,), (1,)), ((), ())),
        preferred_element_type=jnp.float32)
    d2 = u2 - 2.0 * cross + k2

    iota = jax.lax.broadcasted_iota(jnp.int32, (TN, M), 1)
    big = jnp.float32(jnp.inf)

    def argmin_pass(d):
        m = jnp.min(d, axis=-1, keepdims=True)
        eq = d == m
        i = jnp.min(jnp.where(eq, iota, M), axis=-1, keepdims=True)
        d_next = jnp.where(iota == i, big, d)
        return m, i, d_next

    m0, i0, d2a = argmin_pass(d2)
    m1, i1, d2b = argmin_pass(d2a)
    m2, i2, _ = argmin_pass(d2b)

    r0 = 1.0 / (jnp.maximum(m0, 0.0) + 1e-8)
    r1 = 1.0 / (jnp.maximum(m1, 0.0) + 1e-8)
    r2 = 1.0 / (jnp.maximum(m2, 0.0) + 1e-8)
    norm = r0 + r1 + r2
    w0 = r0 / norm
    w1 = r1 / norm
    w2 = r2 / norm

    w = (jnp.where(iota == i0, w0, 0.0)
         + jnp.where(iota == i1, w1, 0.0)
         + jnp.where(iota == i2, w2, 0.0))

    interp = jax.lax.dot_general(
        w, kf, (((1,), (0,)), ((), ())),
        preferred_element_type=jnp.float32)

    out_ref[0, :, :C2] = interp
    out_ref[0, :, C2:] = uf


@jax.jit
def kernel(unknown, known, unknow_feats, known_feats):
    grid = (B, N // TN)
    out = pl.pallas_call(
        _fp_kernel,
        grid=grid,
        in_specs=[
            pl.BlockSpec((1, TN, 3), lambda b, i: (b, i, 0)),
            pl.BlockSpec((1, M, 3), lambda b, i: (b, 0, 0)),
            pl.BlockSpec((1, TN, C1), lambda b, i: (b, i, 0)),
            pl.BlockSpec((1, M, C2), lambda b, i: (b, 0, 0)),
        ],
        out_specs=pl.BlockSpec((1, TN, C1 + C2), lambda b, i: (b, i, 0)),
        out_shape=jax.ShapeDtypeStruct((B, N, C1 + C2), jnp.float32),
    )(unknown, known, unknow_feats, known_feats)
    return (out, out)


# speedup vs baseline: 32.8021x; 1.3230x over previous
import functools

import jax
import jax.numpy as jnp
from jax.experimental import pallas as pl
from jax.experimental.pallas import tpu as pltpu

B, N, M, C1, C2 = 8, 4096, 1024, 64, 128
TN = 512


def _fp_kernel(unknown_ref, known_ref, unknow_feats_ref, known_feats_ref,
               out_ref):
    u = unknown_ref[0]
    k = known_ref[0]
    kf = known_feats_ref[0]
    uf = unknow_feats_ref[0]

    u2 = jnp.sum(u * u, axis=-1, keepdims=True)
    k2 = jnp.sum(k * k, axis=-1)[None, :]
    cross = jax.lax.dot_general(
        -2.0 * u, k, (((1,), (1,)), ((), ())),
        preferred_element_type=jnp.float32)
    d2 = (u2 + cross) + k2

    big = jnp.float32(jnp.inf)

    m0 = jnp.min(d2, axis=-1, keepdims=True)
    eq0 = d2 == m0
    d2a = jnp.where(eq0, big, d2)
    m1 = jnp.min(d2a, axis=-1, keepdims=True)
    eq1 = d2a == m1
    d2b = jnp.where(eq1, big, d2a)
    m2 = jnp.min(d2b, axis=-1, keepdims=True)

    r0 = 1.0 / (jnp.maximum(m0, 0.0) + 1e-8)
    r1 = 1.0 / (jnp.maximum(m1, 0.0) + 1e-8)
    r2 = 1.0 / (jnp.maximum(m2, 0.0) + 1e-8)
    inv_norm = 1.0 / (r0 + r1 + r2)
    w0 = r0 * inv_norm
    w1 = r1 * inv_norm
    w2 = r2 * inv_norm

    w = jnp.where(eq0, w0, jnp.where(eq1, w1, jnp.where(d2b == m2, w2, 0.0)))

    interp = jax.lax.dot_general(
        w, kf, (((1,), (0,)), ((), ())),
        preferred_element_type=jnp.float32)

    out_ref[0, :, :C2] = interp
    out_ref[0, :, C2:] = uf


@jax.jit
def kernel(unknown, known, unknow_feats, known_feats):
    grid = (B, N // TN)
    out = pl.pallas_call(
        _fp_kernel,
        grid=grid,
        in_specs=[
            pl.BlockSpec((1, TN, 3), lambda b, i: (b, i, 0)),
            pl.BlockSpec((1, M, 3), lambda b, i: (b, 0, 0)),
            pl.BlockSpec((1, TN, C1), lambda b, i: (b, i, 0)),
            pl.BlockSpec((1, M, C2), lambda b, i: (b, 0, 0)),
        ],
        out_specs=pl.BlockSpec((1, TN, C1 + C2), lambda b, i: (b, i, 0)),
        out_shape=jax.ShapeDtypeStruct((B, N, C1 + C2), jnp.float32),
    )(unknown, known, unknow_feats, known_feats)
    return (out, out)
